# baseline (speedup 1.0000x reference)
import functools

import jax
import jax.numpy as jnp
from jax import lax
from jax.experimental import pallas as pl
from jax.experimental.pallas import tpu as pltpu
from jax.experimental.pallas import tpu_sc as plsc

VOCAB = 100000
D = 2048
SEQ = 2048
BATCH = 4
ROWS = BATCH * SEQ
NC, NS, LANES = 2, 16, 16
NW = NC * NS
B_PER_W = ROWS // NW
CHUNK = 16
NCHUNK = B_PER_W // CHUNK


def _sc_embed(table, idx2d, pos2d):
    mesh = plsc.VectorSubcoreMesh(core_axis_name="c", subcore_axis_name="s")

    @functools.partial(
        pl.kernel,
        out_type=jax.ShapeDtypeStruct((ROWS, D), jnp.float32),
        mesh=mesh,
        scratch_types=[
            pltpu.VMEM((NCHUNK, CHUNK), jnp.int32),
            pltpu.VMEM((CHUNK, D), jnp.float32),
            pltpu.VMEM((CHUNK, D), jnp.float32),
            pltpu.SemaphoreType.DMA,
        ],
    )
    def k(table_hbm, idx_hbm, pos_hbm, out_hbm, idx_v, rows_v, pos_v, sem):
        wid = lax.axis_index("c") * NS + lax.axis_index("s")
        base = wid * B_PER_W
        pos_base = base % SEQ
        pltpu.sync_copy(idx_hbm.at[pl.ds(wid * NCHUNK, NCHUNK)], idx_v)

        @pl.loop(0, NCHUNK)
        def _(c):
            gather = pltpu.async_copy(table_hbm.at[idx_v.at[c]], rows_v, sem)
            pltpu.sync_copy(pos_hbm.at[pl.ds(pos_base + c * CHUNK, CHUNK)],
                            pos_v)
            gather.wait()

            @pl.loop(0, CHUNK)
            def _(r):
                @pl.loop(0, D, step=LANES)
                def _(col):
                    slc = (pl.ds(r, 1), pl.ds(col, LANES))
                    rows_v.at[*slc][...] = (
                        rows_v.at[*slc][...] + pos_v.at[*slc][...]
                    )

            pltpu.sync_copy(rows_v, out_hbm.at[pl.ds(base + c * CHUNK, CHUNK)])

    return k(table, idx2d, pos2d)


def kernel(x, table, pos_embed):
    idx2d = x.astype(jnp.int32).reshape(NW * NCHUNK, CHUNK)
    pos2d = pos_embed.reshape(SEQ, D)
    out = _sc_embed(table, idx2d, pos2d)
    return out.reshape(BATCH, SEQ, D)


# speedup vs baseline: 1.7117x; 1.7117x over previous
import functools

import jax
import jax.numpy as jnp
from jax import lax
from jax.experimental import pallas as pl
from jax.experimental.pallas import tpu as pltpu
from jax.experimental.pallas import tpu_sc as plsc

VOCAB = 100000
D = 2048
SEQ = 2048
BATCH = 4
ROWS = BATCH * SEQ
NC, NS, LANES = 2, 16, 16
NW = NC * NS
POS_PER_W = SEQ // NW
PGRP = 16
NGRP = POS_PER_W // PGRP
CHUNK = 8
CPG = PGRP * BATCH // CHUNK
NCH = NGRP * CPG


def _sc_embed(table, idx4d, pos2d):
    mesh = plsc.VectorSubcoreMesh(core_axis_name="c", subcore_axis_name="s")

    @functools.partial(
        pl.kernel,
        out_type=jax.ShapeDtypeStruct((ROWS, D), jnp.float32),
        mesh=mesh,
        scratch_types=[
            pltpu.VMEM((NGRP, BATCH, PGRP), jnp.int32),
            pltpu.VMEM((CHUNK, D), jnp.float32),
            pltpu.VMEM((CHUNK, D), jnp.float32),
            pltpu.VMEM((PGRP, D), jnp.float32),
            pltpu.VMEM((PGRP, D), jnp.float32),
            pltpu.SemaphoreType.DMA,
            pltpu.SemaphoreType.DMA,
            pltpu.SemaphoreType.DMA,
            pltpu.SemaphoreType.DMA,
        ],
    )
    def k(table_hbm, idx_hbm, pos_hbm, out_hbm,
          idx_v, rb0, rb1, pb0, pb1, gsem0, gsem1, psem0, psem1):
        rb = (rb0, rb1)
        pb = (pb0, pb1)
        gsem = (gsem0, gsem1)
        psem = (psem0, psem1)
        wid = lax.axis_index("c") * NS + lax.axis_index("s")
        pos0 = wid * POS_PER_W

        def chunk_parts(t, jj):
            g = t // CPG
            b = (jj % CPG) // 2
            h = jj % 2
            return g, b, h

        def gather_descr(t, jj, j):
            g, b, h = chunk_parts(t, jj)
            return pltpu.make_async_copy(
                table_hbm.at[idx_v.at[g, b, pl.ds(h * CHUNK, CHUNK)]],
                rb[j], gsem[j])

        def pos_descr(g, pj):
            return pltpu.make_async_copy(
                pos_hbm.at[pl.ds(pos0 + g * PGRP, PGRP)], pb[pj], psem[pj])

        pltpu.sync_copy(idx_hbm.at[wid], idx_v)
        pos_descr(0, 0).start()
        gather_descr(0, 0, 0).start()
        gather_descr(1, 1, 1).start()

        @pl.loop(0, NCH, step=2 * CPG)
        def _(tt):
            for jj in range(2 * CPG):
                t = tt + jj
                j = jj % 2
                g, b, h = chunk_parts(t, jj)
                pg = (jj // CPG) % 2

                if jj % CPG == 0:
                    pos_descr(g, pg).wait()

                    @pl.when(g < NGRP - 1)
                    def _():
                        pos_descr(g + 1, 1 - pg).start()

                gather_descr(t, jj, j).wait()

                @pl.loop(0, CHUNK)
                def _(r):
                    @pl.loop(0, D, step=LANES, unroll=8)
                    def _(col):
                        rb[j].at[r, pl.ds(col, LANES)][...] = (
                            rb[j].at[r, pl.ds(col, LANES)][...]
                            + pb[pg].at[h * CHUNK + r, pl.ds(col, LANES)][...]
                        )

                out_base = b * SEQ + pos0 + g * PGRP + h * CHUNK
                pltpu.sync_copy(rb[j], out_hbm.at[pl.ds(out_base, CHUNK)])

                @pl.when(t + 2 < NCH)
                def _():
                    t2 = t + 2
                    jj2 = (jj + 2) % (2 * CPG)
                    gather_descr(t2, jj2, j).start()

    return k(table, idx4d, pos2d)


def kernel(x, table, pos_embed):
    idx4d = (x.astype(jnp.int32)
             .reshape(BATCH, NW, NGRP, PGRP)
             .transpose(1, 2, 0, 3))
    pos2d = pos_embed.reshape(SEQ, D)
    out = _sc_embed(table, idx4d, pos2d)
    return out.reshape(BATCH, SEQ, D)


# speedup vs baseline: 2.6899x; 1.5714x over previous
import functools

import jax
import jax.numpy as jnp
from jax import lax
from jax.experimental import pallas as pl
from jax.experimental.pallas import tpu as pltpu
from jax.experimental.pallas import tpu_sc as plsc

VOCAB = 100000
D = 2048
SEQ = 2048
BATCH = 4
ROWS = BATCH * SEQ
NC, NS, LANES = 2, 16, 16
NW = NC * NS
POS_PER_W = SEQ // NW
PGRP = 8
NGRP = POS_PER_W // PGRP
CHUNK = 8
CPG = BATCH
NCH = NGRP * CPG
NRB = 4


def _sc_embed(table, idx4d, pos2d):
    mesh = plsc.VectorSubcoreMesh(core_axis_name="c", subcore_axis_name="s")

    @functools.partial(
        pl.kernel,
        out_type=jax.ShapeDtypeStruct((ROWS, D), jnp.float32),
        mesh=mesh,
        scratch_types=[
            pltpu.VMEM((NGRP, BATCH, PGRP), jnp.int32),
            pltpu.VMEM((NRB, CHUNK, D), jnp.float32),
            pltpu.VMEM((2, PGRP, D), jnp.float32),
            [pltpu.SemaphoreType.DMA] * NRB,
            [pltpu.SemaphoreType.DMA] * NRB,
            [pltpu.SemaphoreType.DMA] * 2,
        ],
    )
    def k(table_hbm, idx_hbm, pos_hbm, out_hbm,
          idx_v, rb, pb, gsem, ssem, psem):
        wid = lax.axis_index("c") * NS + lax.axis_index("s")
        pos0 = wid * POS_PER_W

        def gather_descr(t, j):
            g, b = t // CPG, t % CPG
            return pltpu.make_async_copy(
                table_hbm.at[idx_v.at[g, b]], rb.at[j], gsem[j])

        def store_descr(t, j):
            g, b = t // CPG, t % CPG
            out_base = b * SEQ + pos0 + g * PGRP
            return pltpu.make_async_copy(
                rb.at[j], out_hbm.at[pl.ds(out_base, CHUNK)], ssem[j])

        def pos_descr(g, pj):
            return pltpu.make_async_copy(
                pos_hbm.at[pl.ds(pos0 + g * PGRP, PGRP)], pb.at[pj], psem[pj])

        pltpu.sync_copy(idx_hbm.at[wid], idx_v)
        pos_descr(0, 0).start()
        gather_descr(0, 0).start()
        gather_descr(1, 1).start()

        @pl.loop(0, NCH, step=2 * CPG)
        def _(tt):
            for jj in range(2 * CPG):
                t = tt + jj
                j = jj % NRB
                pg = (jj // CPG) % 2
                g = t // CPG

                if jj % CPG == 0:
                    pos_descr(g, pg).wait()

                    @pl.when(g < NGRP - 1)
                    def _():
                        pos_descr(g + 1, 1 - pg).start()

                gather_descr(t, j).wait()

                @pl.loop(0, CHUNK)
                def _(r):
                    @pl.loop(0, D, step=LANES, unroll=8)
                    def _(col):
                        rb.at[j, r, pl.ds(col, LANES)][...] = (
                            rb.at[j, r, pl.ds(col, LANES)][...]
                            + pb.at[pg, r, pl.ds(col, LANES)][...]
                        )

                store_descr(t, j).start()

                @pl.when(t + 2 < NCH)
                def _():
                    j2 = (jj + 2) % NRB

                    @pl.when(t >= 2)
                    def _():
                        store_descr(t - 2, j2).wait()

                    gather_descr(t + 2, j2).start()

        for jj in range(NRB):
            t = NCH - NRB + jj
            store_descr(t, t % NRB).wait()

    return k(table, idx4d, pos2d)


def kernel(x, table, pos_embed):
    idx4d = (x.astype(jnp.int32)
             .reshape(BATCH, NW, NGRP, PGRP)
             .transpose(1, 2, 0, 3))
    pos2d = pos_embed.reshape(SEQ, D)
    out = _sc_embed(table, idx4d, pos2d)
    return out.reshape(BATCH, SEQ, D)


# speedup vs baseline: 2.8544x; 1.0611x over previous
import functools

import jax
import jax.numpy as jnp
from jax import lax
from jax.experimental import pallas as pl
from jax.experimental.pallas import tpu as pltpu
from jax.experimental.pallas import tpu_sc as plsc

VOCAB = 100000
D = 2048
SEQ = 2048
BATCH = 4
ROWS = BATCH * SEQ
NC, NS, LANES = 2, 16, 16
NW = NC * NS
POS_PER_W = SEQ // NW
PGRP = 8
NGRP = POS_PER_W // PGRP
CHUNK = 8
CPG = BATCH
NCH = NGRP * CPG
NRB = 4


def _sc_embed(table, idx4d, pos2d):
    mesh = plsc.VectorSubcoreMesh(core_axis_name="c", subcore_axis_name="s")

    @functools.partial(
        pl.kernel,
        out_type=jax.ShapeDtypeStruct((ROWS, D), jnp.float32),
        mesh=mesh,
        scratch_types=[
            pltpu.VMEM((NGRP, BATCH, PGRP), jnp.int32),
            pltpu.VMEM((NRB, CHUNK, D), jnp.float32),
            pltpu.VMEM((2, PGRP, D), jnp.float32),
            [pltpu.SemaphoreType.DMA] * NRB,
            [pltpu.SemaphoreType.DMA] * NRB,
            [pltpu.SemaphoreType.DMA] * 2,
        ],
    )
    def k(table_hbm, idx_hbm, pos_hbm, out_hbm,
          idx_v, rb, pb, gsem, ssem, psem):
        wid = lax.axis_index("c") * NS + lax.axis_index("s")
        pos0 = wid * POS_PER_W

        def gather_descr(t, j):
            g, b = t // CPG, t % CPG
            return pltpu.make_async_copy(
                table_hbm.at[idx_v.at[g, b]], rb.at[j], gsem[j])

        def store_descr(t, j):
            g, b = t // CPG, t % CPG
            out_base = b * SEQ + pos0 + g * PGRP
            return pltpu.make_async_copy(
                rb.at[j], out_hbm.at[pl.ds(out_base, CHUNK)], ssem[j])

        def pos_descr(g, pj):
            return pltpu.make_async_copy(
                pos_hbm.at[pl.ds(pos0 + g * PGRP, PGRP)], pb.at[pj], psem[pj])

        pltpu.sync_copy(idx_hbm.at[wid], idx_v)
        pos_descr(0, 0).start()
        gather_descr(0, 0).start()
        gather_descr(1, 1).start()

        @pl.loop(0, NCH, step=2 * CPG)
        def _(tt):
            for jj in range(2 * CPG):
                t = tt + jj
                j = jj % NRB
                pg = (jj // CPG) % 2
                g = t // CPG

                if jj % CPG == 0:
                    pos_descr(g, pg).wait()

                    @pl.when(g < NGRP - 1)
                    def _():
                        pos_descr(g + 1, 1 - pg).start()

                gather_descr(t, j).wait()

                @pl.when(t + 2 < NCH)
                def _():
                    j2 = (jj + 2) % NRB

                    @pl.when(t >= 2)
                    def _():
                        store_descr(t - 2, j2).wait()

                    gather_descr(t + 2, j2).start()

                @pl.loop(0, CHUNK)
                def _(r):
                    @pl.loop(0, D, step=LANES, unroll=16)
                    def _(col):
                        rb.at[j, r, pl.ds(col, LANES)][...] = (
                            rb.at[j, r, pl.ds(col, LANES)][...]
                            + pb.at[pg, r, pl.ds(col, LANES)][...]
                        )

                store_descr(t, j).start()

        for jj in range(NRB):
            t = NCH - NRB + jj
            store_descr(t, t % NRB).wait()

    return k(table, idx4d, pos2d)


def kernel(x, table, pos_embed):
    idx4d = (x.astype(jnp.int32)
             .reshape(BATCH, NW, NGRP, PGRP)
             .transpose(1, 2, 0, 3))
    pos2d = pos_embed.reshape(SEQ, D)
    out = _sc_embed(table, idx4d, pos2d)
    return out.reshape(BATCH, SEQ, D)
